# Initial kernel scaffold; baseline (speedup 1.0000x reference)
import jax
import jax.numpy as jnp
from jax.experimental import pallas as pl


def kernel(x, knn_idx, W_qkv, W_out, b_out):
    raise NotImplementedError("write your pallas kernel here")


# speedup vs baseline: 3.1184x; 3.1184x over previous
import functools

import jax
import jax.numpy as jnp
from jax import lax
from jax.experimental import pallas as pl
from jax.experimental.pallas import tpu as pltpu

NBANKS = 4
RPC = 256


def _qkv_body(d, x_ref, w_ref, q_ref, kv_ref):
    xw = lax.dot_general(x_ref[...], w_ref[...], (((1,), (1,)), ((), ())),
                         preferred_element_type=jnp.float32)
    q_ref[...] = xw[:, :d]
    kv_ref[0] = xw[:, 2 * d:]
    kv_ref[1] = xw[:, d:2 * d]


def _out_body(scale, q_ref, kv_ref, w_ref, b_ref, o_ref):
    v3 = kv_ref[0]
    k3 = kv_ref[1]
    attn = jnp.sum(q_ref[...] * k3, axis=1, keepdims=True) * scale
    h = attn * v3
    o_ref[...] = lax.dot_general(h, w_ref[...], (((1,), (1,)), ((), ())),
                                 preferred_element_type=jnp.float32) + b_ref[...]


def _hops_body(np_rows, d, k, hops, nchunk, kv_ref, dst_ref, out_ref,
               cur, *banks):
    h = pl.program_id(1)
    c = pl.program_id(2)

    @pl.when(c == 0)
    def _start_hop():
        @pl.when(h == 0)
        def _():
            cur[...] = kv_ref[0]

        @pl.when(h > 0)
        def _():
            tot = banks[0][...]
            for b in banks[1:]:
                tot = tot + b[...]
            cur[...] = tot
        for b in banks:
            b[...] = jnp.zeros((np_rows, d), jnp.float32)

    base = c * RPC

    def row_body(r, _):
        row = cur[pl.ds(base + r, 1), :]
        for kk in range(k):
            di = dst_ref[r, kk]
            bnk = banks[kk % NBANKS]
            bnk[pl.ds(di, 1), :] += row
        return 0
    lax.fori_loop(0, RPC, row_body, 0)

    @pl.when((h == hops - 1) & (c == nchunk - 1))
    def _finish():
        tot = banks[0][...]
        for b in banks[1:]:
            tot = tot + b[...]
        out_ref[0] = tot


def kernel(x, knn_idx, W_qkv, W_out, b_out):
    B, N, D = x.shape
    K = knn_idx.shape[-1]
    hops = 3
    BN = B * N
    NP = ((BN + RPC - 1) // RPC) * RPC
    nchunk = NP // RPC

    xp = jnp.pad(x.reshape(BN, D), ((0, NP - BN), (0, 0)))
    batch_off = (jnp.arange(B, dtype=knn_idx.dtype) * N)[:, None, None]
    dst = (knn_idx + batch_off).reshape(BN, K).astype(jnp.int32)
    dst = jnp.pad(dst, ((0, NP - BN), (0, 0)))

    BM = 1024 if NP % 1024 == 0 else RPC
    q, kv = pl.pallas_call(
        functools.partial(_qkv_body, D),
        grid=(NP // BM,),
        in_specs=[pl.BlockSpec((BM, D), lambda i: (i, 0)),
                  pl.BlockSpec((3 * D, D), lambda i: (0, 0))],
        out_specs=[pl.BlockSpec((BM, D), lambda i: (i, 0)),
                   pl.BlockSpec((2, BM, D), lambda i: (0, i, 0))],
        out_shape=[jax.ShapeDtypeStruct((NP, D), jnp.float32),
                   jax.ShapeDtypeStruct((2, NP, D), jnp.float32)],
    )(xp, W_qkv)

    kv3 = pl.pallas_call(
        functools.partial(_hops_body, NP, D, K, hops, nchunk),
        grid=(2, hops, nchunk),
        in_specs=[pl.BlockSpec((1, NP, D), lambda t, h, c: (t, 0, 0)),
                  pl.BlockSpec((RPC, K), lambda t, h, c: (c, 0),
                               memory_space=pltpu.SMEM)],
        out_specs=pl.BlockSpec((1, NP, D), lambda t, h, c: (t, 0, 0)),
        out_shape=jax.ShapeDtypeStruct((2, NP, D), jnp.float32),
        scratch_shapes=[pltpu.VMEM((NP, D), jnp.float32)] * (1 + NBANKS),
    )(kv, dst)

    scale = float((1.0 / (K + 1e-6)) ** (2 * hops))
    out = pl.pallas_call(
        functools.partial(_out_body, scale),
        grid=(NP // BM,),
        in_specs=[pl.BlockSpec((BM, D), lambda i: (i, 0)),
                  pl.BlockSpec((2, BM, D), lambda i: (0, i, 0)),
                  pl.BlockSpec((D, D), lambda i: (0, 0)),
                  pl.BlockSpec((1, D), lambda i: (0, 0))],
        out_specs=pl.BlockSpec((BM, D), lambda i: (i, 0)),
        out_shape=jax.ShapeDtypeStruct((NP, D), jnp.float32),
    )(q, kv3, W_out, b_out.reshape(1, D))

    return out[:BN].reshape(B, N, D)


# speedup vs baseline: 4.8494x; 1.5551x over previous
import functools

import jax
import jax.numpy as jnp
from jax import lax
from jax.experimental import pallas as pl
from jax.experimental.pallas import tpu as pltpu
from jax.experimental.pallas import tpu_sc as plsc

NUM_CORES = 2
NUM_SUBCORES = 16
NT = NUM_CORES * NUM_SUBCORES
GC = 64


def _qkv_body(d, x_ref, w_ref, q_ref, kv_ref):
    xw = lax.dot_general(x_ref[...], w_ref[...], (((1,), (1,)), ((), ())),
                         preferred_element_type=jnp.float32)
    q_ref[...] = xw[:, :d]
    kv_ref[0] = xw[:, 2 * d:]
    kv_ref[1] = xw[:, d:2 * d]


def _out_body(scale, q_ref, kv_ref, w_ref, b_ref, o_ref):
    v3 = kv_ref[0]
    k3 = kv_ref[1]
    attn = jnp.sum(q_ref[...] * k3, axis=1, keepdims=True) * scale
    h = attn * v3
    o_ref[...] = lax.dot_general(h, w_ref[...], (((1,), (1,)), ((), ())),
                                 preferred_element_type=jnp.float32) + b_ref[...]


def _make_sc_hops(np_rows, d, k, hops):
    cpt = d // NT
    npg = np_rows // 16
    nchunks = npg // GC
    fl = cpt * np_rows
    mesh = plsc.VectorSubcoreMesh(core_axis_name="c", subcore_axis_name="s",
                                  num_cores=NUM_CORES,
                                  num_subcores=NUM_SUBCORES)

    @functools.partial(
        pl.kernel,
        out_type=jax.ShapeDtypeStruct((2, NT, fl), jnp.float32),
        mesh=mesh,
        compiler_params=pltpu.CompilerParams(needs_layout_passes=False),
        scratch_types=[
            pltpu.VMEM((fl,), jnp.float32),
            pltpu.VMEM((fl,), jnp.float32),
            pltpu.VMEM((GC * k * 16,), jnp.int32),
        ],
    )
    def sc_hops(kv_hbm, idx_hbm, out_hbm, bufa, bufb, idxb):
        w = lax.axis_index("c") * NUM_SUBCORES + lax.axis_index("s")
        zero16 = jnp.zeros((16,), jnp.float32)
        for t in range(2):
            pltpu.sync_copy(kv_hbm.at[t, w], bufa)
            for h in range(hops):
                src, dst = (bufa, bufb) if h % 2 == 0 else (bufb, bufa)

                def zb(i, _, dst=dst):
                    dst[pl.ds(i * 16, 16)] = zero16
                    return 0
                lax.fori_loop(0, fl // 16, zb, 0)

                def chunk_body(ch, _, src=src, dst=dst):
                    pltpu.sync_copy(
                        idx_hbm.at[pl.ds(ch * (GC * k * 16), GC * k * 16)],
                        idxb)

                    def g_body(g, _):
                        gg = ch * GC + g
                        vals = [src[pl.ds(c * np_rows + gg * 16, 16)]
                                for c in range(cpt)]
                        for kk in range(k):
                            dsts = idxb[pl.ds((g * k + kk) * 16, 16)]
                            for c in range(cpt):
                                plsc.addupdate_scatter(
                                    dst, [dsts + (c * np_rows)], vals[c])
                        return 0
                    lax.fori_loop(0, GC, g_body, 0)
                    return 0
                lax.fori_loop(0, nchunks, chunk_body, 0)
            final = bufb if hops % 2 == 1 else bufa
            pltpu.sync_copy(final, out_hbm.at[t, w])

    return sc_hops


def kernel(x, knn_idx, W_qkv, W_out, b_out):
    B, N, D = x.shape
    K = knn_idx.shape[-1]
    hops = 3
    BN = B * N
    gran = 16 * GC
    NP = ((BN + gran - 1) // gran) * gran
    npg = NP // 16

    xp = jnp.pad(x.reshape(BN, D), ((0, NP - BN), (0, 0)))
    batch_off = (jnp.arange(B, dtype=knn_idx.dtype) * N)[:, None, None]
    dst = (knn_idx + batch_off).reshape(BN, K).astype(jnp.int32)
    dst = jnp.pad(dst, ((0, NP - BN), (0, 0)))
    idx_flat = dst.reshape(npg, 16, K).transpose(0, 2, 1).reshape(-1)

    BM = 1024
    q, kv = pl.pallas_call(
        functools.partial(_qkv_body, D),
        grid=(NP // BM,),
        in_specs=[pl.BlockSpec((BM, D), lambda i: (i, 0)),
                  pl.BlockSpec((3 * D, D), lambda i: (0, 0))],
        out_specs=[pl.BlockSpec((BM, D), lambda i: (i, 0)),
                   pl.BlockSpec((2, BM, D), lambda i: (0, i, 0))],
        out_shape=[jax.ShapeDtypeStruct((NP, D), jnp.float32),
                   jax.ShapeDtypeStruct((2, NP, D), jnp.float32)],
    )(xp, W_qkv)

    cpt = D // NT
    kv_planes = kv.reshape(2, NP, NT, cpt).transpose(0, 2, 3, 1)
    kv_planes = kv_planes.reshape(2, NT, cpt * NP)
    out_planes = _make_sc_hops(NP, D, K, hops)(kv_planes, idx_flat)
    kv3 = out_planes.reshape(2, NT, cpt, NP).transpose(0, 3, 1, 2)
    kv3 = kv3.reshape(2, NP, D)

    scale = float((1.0 / (K + 1e-6)) ** (2 * hops))
    out = pl.pallas_call(
        functools.partial(_out_body, scale),
        grid=(NP // BM,),
        in_specs=[pl.BlockSpec((BM, D), lambda i: (i, 0)),
                  pl.BlockSpec((2, BM, D), lambda i: (0, i, 0)),
                  pl.BlockSpec((D, D), lambda i: (0, 0)),
                  pl.BlockSpec((1, D), lambda i: (0, 0))],
        out_specs=pl.BlockSpec((BM, D), lambda i: (i, 0)),
        out_shape=jax.ShapeDtypeStruct((NP, D), jnp.float32),
    )(q, kv3, W_out, b_out.reshape(1, D))

    return out[:BN].reshape(B, N, D)


# speedup vs baseline: 6.8081x; 1.4039x over previous
import functools

import jax
import jax.numpy as jnp
from jax import lax
from jax.experimental import pallas as pl
from jax.experimental.pallas import tpu as pltpu
from jax.experimental.pallas import tpu_sc as plsc

NUM_CORES = 2
NUM_SUBCORES = 16
NT = NUM_CORES * NUM_SUBCORES
GC = 64


def _qkv_body(d, x_ref, w_ref, q_ref, kv_ref):
    xw = lax.dot_general(x_ref[...], w_ref[...], (((1,), (1,)), ((), ())),
                         preferred_element_type=jnp.float32)
    q_ref[...] = xw[:, :d]
    kv_ref[0] = xw[:, 2 * d:]
    kv_ref[1] = xw[:, d:2 * d]


def _out_body(scale, q_ref, kv_ref, w_ref, b_ref, o_ref):
    v3 = kv_ref[0]
    k3 = kv_ref[1]
    attn = jnp.sum(q_ref[...] * k3, axis=1, keepdims=True) * scale
    h = attn * v3
    o_ref[...] = lax.dot_general(h, w_ref[...], (((1,), (1,)), ((), ())),
                                 preferred_element_type=jnp.float32) + b_ref[...]


def _make_sc_hops(np_rows, d, k, hops):
    cpt = d // NT
    npg = np_rows // 16
    nchunks = npg // GC
    fl = cpt * np_rows
    mesh = plsc.VectorSubcoreMesh(core_axis_name="c", subcore_axis_name="s",
                                  num_cores=NUM_CORES,
                                  num_subcores=NUM_SUBCORES)

    @functools.partial(
        pl.kernel,
        out_type=jax.ShapeDtypeStruct((2, NT, fl), jnp.float32),
        mesh=mesh,
        compiler_params=pltpu.CompilerParams(needs_layout_passes=False),
        scratch_types=[
            pltpu.VMEM((fl,), jnp.float32),
            pltpu.VMEM((fl,), jnp.float32),
            pltpu.VMEM((GC * k * 16,), jnp.int32),
        ],
    )
    def sc_hops(kv_hbm, idx_hbm, out_hbm, bufa, bufb, idxb):
        w = lax.axis_index("c") * NUM_SUBCORES + lax.axis_index("s")
        zero16 = jnp.zeros((16,), jnp.float32)
        for t in range(2):
            pltpu.sync_copy(kv_hbm.at[t, w], bufa)
            for h in range(hops):
                src, dst = (bufa, bufb) if h % 2 == 0 else (bufb, bufa)

                def zb(i, _, dst=dst):
                    dst[pl.ds(i * 16, 16)] = zero16
                    return 0
                lax.fori_loop(0, fl // 16, zb, 0)

                def chunk_body(ch, _, src=src, dst=dst):
                    pltpu.sync_copy(
                        idx_hbm.at[pl.ds(ch * (GC * k * 16), GC * k * 16)],
                        idxb)

                    @plsc.parallel_loop(0, GC, 1)
                    def g_body(g, src=src, dst=dst, ch=ch):
                        gg = ch * GC + g
                        vals = [src[pl.ds(c * np_rows + gg * 16, 16)]
                                for c in range(cpt)]
                        d0 = idxb[pl.ds((g * k) * 16, 16)]
                        idxs = [d0 + (c * np_rows) for c in range(cpt)]
                        for kk in range(k):
                            if kk + 1 < k:
                                dn = idxb[pl.ds((g * k + kk + 1) * 16, 16)]
                                nxt = [dn + (c * np_rows) for c in range(cpt)]
                            for c in range(cpt):
                                plsc.addupdate_scatter(dst, [idxs[c]], vals[c])
                            if kk + 1 < k:
                                idxs = nxt
                    return 0
                lax.fori_loop(0, nchunks, chunk_body, 0)
            final = bufb if hops % 2 == 1 else bufa
            pltpu.sync_copy(final, out_hbm.at[t, w])

    return sc_hops


def kernel(x, knn_idx, W_qkv, W_out, b_out):
    B, N, D = x.shape
    K = knn_idx.shape[-1]
    hops = 3
    BN = B * N
    gran = 16 * GC
    NP = ((BN + gran - 1) // gran) * gran
    npg = NP // 16

    xp = jnp.pad(x.reshape(BN, D), ((0, NP - BN), (0, 0)))
    batch_off = (jnp.arange(B, dtype=knn_idx.dtype) * N)[:, None, None]
    dst = (knn_idx + batch_off).reshape(BN, K).astype(jnp.int32)
    dst = jnp.pad(dst, ((0, NP - BN), (0, 0)))
    idx_flat = dst.reshape(npg, 16, K).transpose(0, 2, 1).reshape(-1)

    BM = 1024
    q, kv = pl.pallas_call(
        functools.partial(_qkv_body, D),
        grid=(NP // BM,),
        in_specs=[pl.BlockSpec((BM, D), lambda i: (i, 0)),
                  pl.BlockSpec((3 * D, D), lambda i: (0, 0))],
        out_specs=[pl.BlockSpec((BM, D), lambda i: (i, 0)),
                   pl.BlockSpec((2, BM, D), lambda i: (0, i, 0))],
        out_shape=[jax.ShapeDtypeStruct((NP, D), jnp.float32),
                   jax.ShapeDtypeStruct((2, NP, D), jnp.float32)],
    )(xp, W_qkv)

    cpt = D // NT
    kv_planes = kv.reshape(2, NP, NT, cpt).transpose(0, 2, 3, 1)
    kv_planes = kv_planes.reshape(2, NT, cpt * NP)
    out_planes = _make_sc_hops(NP, D, K, hops)(kv_planes, idx_flat)
    kv3 = out_planes.reshape(2, NT, cpt, NP).transpose(0, 3, 1, 2)
    kv3 = kv3.reshape(2, NP, D)

    scale = float((1.0 / (K + 1e-6)) ** (2 * hops))
    out = pl.pallas_call(
        functools.partial(_out_body, scale),
        grid=(NP // BM,),
        in_specs=[pl.BlockSpec((BM, D), lambda i: (i, 0)),
                  pl.BlockSpec((2, BM, D), lambda i: (0, i, 0)),
                  pl.BlockSpec((D, D), lambda i: (0, 0)),
                  pl.BlockSpec((1, D), lambda i: (0, 0))],
        out_specs=pl.BlockSpec((BM, D), lambda i: (i, 0)),
        out_shape=jax.ShapeDtypeStruct((NP, D), jnp.float32),
    )(q, kv3, W_out, b_out.reshape(1, D))

    return out[:BN].reshape(B, N, D)


# speedup vs baseline: 7.4441x; 1.0934x over previous
import functools

import jax
import jax.numpy as jnp
from jax import lax
from jax.experimental import pallas as pl
from jax.experimental.pallas import tpu as pltpu
from jax.experimental.pallas import tpu_sc as plsc

NUM_CORES = 2
NUM_SUBCORES = 16
NT = NUM_CORES * NUM_SUBCORES
GC = 32


def _qkv_body(d, x_ref, w_ref, q_ref, kv_ref):
    xw = lax.dot_general(x_ref[...], w_ref[...], (((1,), (1,)), ((), ())),
                         preferred_element_type=jnp.float32)
    q_ref[...] = xw[:, :d]
    kv_ref[0] = xw[:, 2 * d:]
    kv_ref[1] = xw[:, d:2 * d]


def _out_body(scale, q_ref, kv_ref, w_ref, b_ref, o_ref):
    v3 = kv_ref[0]
    k3 = kv_ref[1]
    attn = jnp.sum(q_ref[...] * k3, axis=1, keepdims=True) * scale
    h = attn * v3
    o_ref[...] = lax.dot_general(h, w_ref[...], (((1,), (1,)), ((), ())),
                                 preferred_element_type=jnp.float32) + b_ref[...]


def _make_sc_hops(np_rows, d, k, hops):
    cpt = d // NT
    npg = np_rows // 16
    nchunks = npg // GC
    fl = cpt * np_rows
    mesh = plsc.VectorSubcoreMesh(core_axis_name="c", subcore_axis_name="s",
                                  num_cores=NUM_CORES,
                                  num_subcores=NUM_SUBCORES)

    @functools.partial(
        pl.kernel,
        out_type=jax.ShapeDtypeStruct((2, NT, fl), jnp.float32),
        mesh=mesh,
        compiler_params=pltpu.CompilerParams(needs_layout_passes=False),
        scratch_types=[
            pltpu.VMEM((fl,), jnp.float32),
            pltpu.VMEM((fl,), jnp.float32),
            pltpu.VMEM((GC * k * 16,), jnp.int32),
            pltpu.VMEM((GC * k * 16,), jnp.int32),
            pltpu.SemaphoreType.DMA,
            pltpu.SemaphoreType.DMA,
        ],
    )
    def sc_hops(kv_hbm, idx_hbm, out_hbm, bufa, bufb, idxb0, idxb1, sem0,
                sem1):
        w = lax.axis_index("c") * NUM_SUBCORES + lax.axis_index("s")
        zero16 = jnp.zeros((16,), jnp.float32)
        L = GC * k * 16

        def groups(src, dst, ib, ch):
            @plsc.parallel_loop(0, GC, 1)
            def g_body(g):
                gg = ch * GC + g
                vals = [src[pl.ds(c * np_rows + gg * 16, 16)]
                        for c in range(cpt)]
                d0 = ib[pl.ds((g * k) * 16, 16)]
                idxs = [d0 + (c * np_rows) for c in range(cpt)]
                for kk in range(k):
                    if kk + 1 < k:
                        dn = ib[pl.ds((g * k + kk + 1) * 16, 16)]
                        nxt = [dn + (c * np_rows) for c in range(cpt)]
                    for c in range(cpt):
                        plsc.addupdate_scatter(dst, [idxs[c]], vals[c])
                    if kk + 1 < k:
                        idxs = nxt

        pltpu.sync_copy(idx_hbm.at[pl.ds(0, L)], idxb0)
        for t in range(2):
            pltpu.sync_copy(kv_hbm.at[t, w], bufa)
            for h in range(hops):
                src, dst = (bufa, bufb) if h % 2 == 0 else (bufb, bufa)

                def zb(i, _, dst=dst):
                    dst[pl.ds(i * 16, 16)] = zero16
                    return 0
                lax.fori_loop(0, fl // 16, zb, 0)

                def pair_body(p, _, src=src, dst=dst):
                    ch0 = p * 2
                    cp1 = pltpu.async_copy(
                        idx_hbm.at[pl.ds((ch0 + 1) * L, L)], idxb1, sem1)
                    groups(src, dst, idxb0, ch0)
                    cp1.wait()
                    cp0 = pltpu.async_copy(
                        idx_hbm.at[pl.ds(((ch0 + 2) % nchunks) * L, L)],
                        idxb0, sem0)
                    groups(src, dst, idxb1, ch0 + 1)
                    cp0.wait()
                    return 0
                lax.fori_loop(0, nchunks // 2, pair_body, 0)
            final = bufb if hops % 2 == 1 else bufa
            pltpu.sync_copy(final, out_hbm.at[t, w])

    return sc_hops


def kernel(x, knn_idx, W_qkv, W_out, b_out):
    B, N, D = x.shape
    K = knn_idx.shape[-1]
    hops = 3
    BN = B * N
    gran = 16 * GC
    NP = ((BN + gran - 1) // gran) * gran
    npg = NP // 16

    xp = jnp.pad(x.reshape(BN, D), ((0, NP - BN), (0, 0)))
    batch_off = (jnp.arange(B, dtype=knn_idx.dtype) * N)[:, None, None]
    dst = (knn_idx + batch_off).reshape(BN, K).astype(jnp.int32)
    dst = jnp.pad(dst, ((0, NP - BN), (0, 0)))
    idx_flat = dst.reshape(npg, 16, K).transpose(0, 2, 1).reshape(-1)

    BM = 1024
    q, kv = pl.pallas_call(
        functools.partial(_qkv_body, D),
        grid=(NP // BM,),
        in_specs=[pl.BlockSpec((BM, D), lambda i: (i, 0)),
                  pl.BlockSpec((3 * D, D), lambda i: (0, 0))],
        out_specs=[pl.BlockSpec((BM, D), lambda i: (i, 0)),
                   pl.BlockSpec((2, BM, D), lambda i: (0, i, 0))],
        out_shape=[jax.ShapeDtypeStruct((NP, D), jnp.float32),
                   jax.ShapeDtypeStruct((2, NP, D), jnp.float32)],
    )(xp, W_qkv)

    cpt = D // NT
    kv_planes = kv.reshape(2, NP, NT, cpt).transpose(0, 2, 3, 1)
    kv_planes = kv_planes.reshape(2, NT, cpt * NP)
    out_planes = _make_sc_hops(NP, D, K, hops)(kv_planes, idx_flat)
    kv3 = out_planes.reshape(2, NT, cpt, NP).transpose(0, 3, 1, 2)
    kv3 = kv3.reshape(2, NP, D)

    scale = float((1.0 / (K + 1e-6)) ** (2 * hops))
    out = pl.pallas_call(
        functools.partial(_out_body, scale),
        grid=(NP // BM,),
        in_specs=[pl.BlockSpec((BM, D), lambda i: (i, 0)),
                  pl.BlockSpec((2, BM, D), lambda i: (0, i, 0)),
                  pl.BlockSpec((D, D), lambda i: (0, 0)),
                  pl.BlockSpec((1, D), lambda i: (0, 0))],
        out_specs=pl.BlockSpec((BM, D), lambda i: (i, 0)),
        out_shape=jax.ShapeDtypeStruct((NP, D), jnp.float32),
    )(q, kv3, W_out, b_out.reshape(1, D))

    return out[:BN].reshape(B, N, D)
